# baseline (speedup 1.0000x reference)
import jax
import jax.numpy as jnp
from jax.experimental import pallas as pl


def _pick_block(n, candidates=(512, 400, 256, 200, 80, 16, 8, 1)):
    for c in candidates:
        if n % c == 0:
            return c
    return n


def _mm_body(x_ref, w_ref, o_ref):
    o_ref[...] = jnp.dot(x_ref[...], w_ref[...],
                         preferred_element_type=jnp.float32)


def _matmul(x, w):
    n, k = x.shape
    m = w.shape[1]
    bn = _pick_block(n, (1000, 800, 500, 400, 200, 100, 8, 1))
    return pl.pallas_call(
        _mm_body,
        grid=(n // bn,),
        in_specs=[
            pl.BlockSpec((bn, k), lambda i: (i, 0)),
            pl.BlockSpec((k, m), lambda i: (0, 0)),
        ],
        out_specs=pl.BlockSpec((bn, m), lambda i: (i, 0)),
        out_shape=jax.ShapeDtypeStruct((n, m), jnp.float32),
    )(x, w)


def _layer_body(a_ref, s_ref, b_ref, w_ref, c_ref, o_ref):
    h = jnp.tanh(jnp.dot(a_ref[...], s_ref[...],
                         preferred_element_type=jnp.float32) + b_ref[...])
    o_ref[...] = jnp.dot(h, w_ref[...],
                         preferred_element_type=jnp.float32) + c_ref[...]


def _gcn_layer(adj, s, b, w, c, bn):
    n, k = adj.shape
    h = s.shape[1]
    m = w.shape[1]
    return pl.pallas_call(
        _layer_body,
        grid=(n // bn,),
        in_specs=[
            pl.BlockSpec((bn, k), lambda i: (i, 0)),
            pl.BlockSpec((k, h), lambda i: (0, 0)),
            pl.BlockSpec((1, h), lambda i: (0, 0)),
            pl.BlockSpec((h, m), lambda i: (0, 0)),
            pl.BlockSpec((1, m), lambda i: (0, 0)),
        ],
        out_specs=pl.BlockSpec((bn, m), lambda i: (i, 0)),
        out_shape=jax.ShapeDtypeStruct((n, m), jnp.float32),
    )(adj, s, b, w, c)


def _log_softmax(x):
    s = x - jnp.max(x, axis=-1, keepdims=True)
    return s - jnp.log(jnp.sum(jnp.exp(s), axis=-1, keepdims=True))


def _epi_body(xa1_ref, xa2_ref, xp1_ref, xp2_ref, wa_ref, wb_ref, fb_ref,
              o1_ref, o2_ref, o3_ref):
    fused = (jnp.dot(xa1_ref[...], wa_ref[...],
                     preferred_element_type=jnp.float32)
             + jnp.dot(xa2_ref[...], wb_ref[...],
                       preferred_element_type=jnp.float32)
             + fb_ref[...])
    o1_ref[...] = _log_softmax(fused)
    o2_ref[...] = _log_softmax(xp1_ref[...])
    o3_ref[...] = _log_softmax(xp2_ref[...])


def _epilogue(x_a1, x_a2, x_p1, x_p2, w_a, w_b, fb):
    n, m = x_a1.shape
    bn = _pick_block(n, (1000, 800, 500, 400, 200, 100, 8, 1))
    io_spec = pl.BlockSpec((bn, m), lambda i: (i, 0))
    w_spec = pl.BlockSpec((m, m), lambda i: (0, 0))
    out_sds = jax.ShapeDtypeStruct((n, m), jnp.float32)
    return pl.pallas_call(
        _epi_body,
        grid=(n // bn,),
        in_specs=[io_spec, io_spec, io_spec, io_spec, w_spec, w_spec,
                  pl.BlockSpec((1, m), lambda i: (0, 0))],
        out_specs=[io_spec, io_spec, io_spec],
        out_shape=[out_sds, out_sds, out_sds],
    )(x_a1, x_a2, x_p1, x_p2, w_a, w_b, fb)


def kernel(x, A1, P1, A2, P2, params):
    n = x.shape[0]
    nhid = params['W_A1'][0].shape[1]
    bn = _pick_block(n)

    order = ['A1', 'P1', 'A2', 'P2']
    adjs = {'A1': A1, 'P1': P1, 'A2': A2, 'P2': P2}

    w1cat = jnp.concatenate([params['W_' + br][0] for br in order], axis=1)
    s0cat = _matmul(x, w1cat)

    heads = {}
    for j, br in enumerate(order):
        s0 = s0cat[:, j * nhid:(j + 1) * nhid]
        b1 = params['b_' + br][0].reshape(1, -1)
        b2 = params['b_' + br][1].reshape(1, -1)
        w2 = params['W_' + br][1]
        lin_w = params['lin_' + br + '_W']
        lin_b = params['lin_' + br + '_b'].reshape(1, -1)
        zero = jnp.zeros_like(b2)
        s1 = _gcn_layer(adjs[br], s0, b1, w2, zero, bn)
        heads[br] = _gcn_layer(adjs[br], s1, b2, lin_w, lin_b, bn)

    nclass = heads['A1'].shape[1]
    w_a = params['fusion_W'][:nclass]
    w_b = params['fusion_W'][nclass:]
    fb = params['fusion_b'].reshape(1, -1)
    o1, o2, o3 = _epilogue(heads['A1'], heads['A2'], heads['P1'], heads['P2'],
                           w_a, w_b, fb)
    return (o1, o2, o3, heads['A1'])
